# baseline (speedup 1.0000x reference)
import jax
import jax.numpy as jnp
from jax.experimental import pallas as pl
from jax.experimental.pallas import tpu as pltpu

_LEVEL = 15.0
_EPS = 0.001
_HALF_L = (_LEVEL - 1.0) * (1.0 + _EPS) / 2.0
_HALF_W = 7.0


def _fsq_block(z_ref, wct_ref, bc_ref, wet_ref, be_ref, traw_ref,
               zq_ref, err_ref):
    t = jax.nn.softplus(traw_ref[...])
    z = z_ref[...]
    zc = jnp.dot(z, wct_ref[...],
                 preferred_element_type=jnp.float32) + bc_ref[...]
    z_bound = jnp.tanh(zc / t) * _HALF_L
    k = jnp.round(z_bound)
    codes = k * (t * (1.0 / _HALF_W))
    diff = zc - codes
    err_ref[...] = jnp.full((1, 1, 1), jnp.sum(diff * diff), jnp.float32)

    we_scaled = (wet_ref[...] * (t.reshape(3, 1) * (1.0 / _HALF_W)))
    zq_ref[...] = jnp.dot(k.astype(jnp.bfloat16),
                          we_scaled.astype(jnp.bfloat16),
                          preferred_element_type=jnp.float32) + be_ref[...]


def kernel(z, W_c, b_c, W_e, b_e, T_raw):
    B, S, D = z.shape
    rows = B * S
    z2 = z.reshape(rows, D)
    blk = 512
    grid = rows // blk

    zq2, err = pl.pallas_call(
        _fsq_block,
        grid=(grid,),
        in_specs=[
            pl.BlockSpec((blk, D), lambda i: (i, 0)),
            pl.BlockSpec((D, 3), lambda i: (0, 0)),
            pl.BlockSpec((1, 3), lambda i: (0, 0)),
            pl.BlockSpec((3, D), lambda i: (0, 0)),
            pl.BlockSpec((1, D), lambda i: (0, 0)),
            pl.BlockSpec((1, 3), lambda i: (0, 0)),
        ],
        out_specs=[
            pl.BlockSpec((blk, D), lambda i: (i, 0)),
            pl.BlockSpec((1, 1, 1), lambda i: (i, 0, 0)),
        ],
        out_shape=[
            jax.ShapeDtypeStruct((rows, D), jnp.float32),
            jax.ShapeDtypeStruct((grid, 1, 1), jnp.float32),
        ],
        compiler_params=pltpu.CompilerParams(
            dimension_semantics=("parallel",),
        ),
    )(z2, W_c.T, b_c.reshape(1, 3), W_e.T, b_e.reshape(1, D),
      T_raw.reshape(1, 3))

    z_q = zq2.reshape(B, S, D)
    quantization_error = jnp.sum(err) / (rows * 3)
    return (z_q, quantization_error)


# speedup vs baseline: 1.3251x; 1.3251x over previous
import jax
import jax.numpy as jnp
from jax.experimental import pallas as pl
from jax.experimental.pallas import tpu as pltpu

_LEVEL = 15.0
_EPS = 0.001
_HALF_L = (_LEVEL - 1.0) * (1.0 + _EPS) / 2.0
_HALF_W = 7.0


def _fsq_block(z_ref, wct_ref, bc_ref, wet_ref, be_ref, traw_ref,
               zq_ref, err_ref):
    t = jax.nn.softplus(traw_ref[...])
    z = z_ref[...]
    zc = jnp.dot(z, wct_ref[...],
                 preferred_element_type=jnp.float32) + bc_ref[...]
    z_bound = jnp.tanh(zc / t) * _HALF_L
    k = jnp.round(z_bound)
    codes = k * (t * (1.0 / _HALF_W))
    diff = zc - codes
    err_ref[...] = jnp.full((1, 1, 1), jnp.sum(diff * diff), jnp.float32)

    we_scaled = (wet_ref[...] * (t.reshape(3, 1) * (1.0 / _HALF_W)))
    zq_ref[...] = jnp.dot(k.astype(jnp.bfloat16),
                          we_scaled.astype(jnp.bfloat16),
                          preferred_element_type=jnp.float32) + be_ref[...]


def kernel(z, W_c, b_c, W_e, b_e, T_raw):
    B, S, D = z.shape
    rows = B * S
    z2 = z.reshape(rows, D)
    blk = 2048
    grid = rows // blk

    zq2, err = pl.pallas_call(
        _fsq_block,
        grid=(grid,),
        in_specs=[
            pl.BlockSpec((blk, D), lambda i: (i, 0)),
            pl.BlockSpec((D, 3), lambda i: (0, 0)),
            pl.BlockSpec((1, 3), lambda i: (0, 0)),
            pl.BlockSpec((3, D), lambda i: (0, 0)),
            pl.BlockSpec((1, D), lambda i: (0, 0)),
            pl.BlockSpec((1, 3), lambda i: (0, 0)),
        ],
        out_specs=[
            pl.BlockSpec((blk, D), lambda i: (i, 0)),
            pl.BlockSpec((1, 1, 1), lambda i: (i, 0, 0)),
        ],
        out_shape=[
            jax.ShapeDtypeStruct((rows, D), jnp.float32),
            jax.ShapeDtypeStruct((grid, 1, 1), jnp.float32),
        ],
        compiler_params=pltpu.CompilerParams(
            dimension_semantics=("parallel",),
        ),
    )(z2, W_c.T, b_c.reshape(1, 3), W_e.T, b_e.reshape(1, D),
      T_raw.reshape(1, 3))

    z_q = zq2.reshape(B, S, D)
    quantization_error = jnp.sum(err) / (rows * 3)
    return (z_q, quantization_error)
